# baseline (speedup 1.0000x reference)
import functools

import jax
import jax.numpy as jnp
import numpy as np
from jax import lax
from jax.experimental import pallas as pl
from jax.experimental.pallas import tpu as pltpu

_NUM_CLASSES = 100000
_NUM_SAMPLED = 8192
_NUM_TRUE = 3
_DIM = 128
_BATCH = 1024

_BN = 2048
_NBLK = pl.cdiv(_NUM_CLASSES, _BN)


def _log_expected_count(ids_f32):
    p = (jnp.log(ids_f32 + 2.0) - jnp.log(ids_f32 + 1.0)) / jnp.log(
        jnp.float32(_NUM_CLASSES + 1.0))
    return jnp.log(1.0 - jnp.exp(_NUM_SAMPLED * jnp.log(1.0 - p)))


def _proj_body(pred_ref, w_ref, b_ref, wv_ref, out_ref, s_ref, acc_ref):
    k = pl.program_id(0)
    tile = (
        jnp.dot(pred_ref[...], w_ref[...], preferred_element_type=jnp.float32)
        + b_ref[...])
    out_ref[...] = tile
    weighted = wv_ref[...] * jnp.exp(tile)

    @pl.when(k == 0)
    def _init():
        acc_ref[...] = jnp.zeros_like(acc_ref)

    @pl.when(k < _NBLK - 1)
    def _accum():
        acc_ref[...] += jnp.sum(weighted, axis=1, keepdims=True)

    @pl.when(k == _NBLK - 1)
    def _accum_tail():
        lane = jax.lax.broadcasted_iota(jnp.int32, (1, _BN), 1)
        valid = lane < (_NUM_CLASSES - (_NBLK - 1) * _BN)
        safe = jnp.where(valid, weighted, 0.0)
        acc_ref[...] += jnp.sum(safe, axis=1, keepdims=True)
        s_ref[...] = acc_ref[...]


def _projection(pred, w, bias2d, wvec2d):
    return pl.pallas_call(
        _proj_body,
        grid=(_NBLK,),
        in_specs=[
            pl.BlockSpec((_BATCH, _DIM), lambda k: (0, 0)),
            pl.BlockSpec((_DIM, _BN), lambda k: (0, k)),
            pl.BlockSpec((1, _BN), lambda k: (0, k)),
            pl.BlockSpec((1, _BN), lambda k: (0, k)),
        ],
        out_specs=[
            pl.BlockSpec((_BATCH, _BN), lambda k: (0, k)),
            pl.BlockSpec((_BATCH, 1), lambda k: (0, 0)),
        ],
        out_shape=[
            jax.ShapeDtypeStruct((_BATCH, _NUM_CLASSES), jnp.float32),
            jax.ShapeDtypeStruct((_BATCH, 1), jnp.float32),
        ],
        scratch_shapes=[pltpu.VMEM((_BATCH, 1), jnp.float32)],
    )(pred, w, bias2d, wvec2d)


def _loss_body(s_ref, tgt_ref, tgtv_ref, out_hbm, loss_ref, out_alias_ref,
               tile_ref, sem):
    del out_alias_ref
    n = _BATCH * _NUM_TRUE

    def issue(i, _):
        b = i // _NUM_TRUE
        t = i - b * _NUM_TRUE
        cls = tgt_ref[b, t]
        rb = pl.multiple_of((b // 8) * 8, 8)
        cc = pl.multiple_of(
            jnp.minimum((cls // 128) * 128, _NUM_CLASSES - 128), 128)
        pltpu.make_async_copy(
            out_hbm.at[pl.ds(rb, 8), pl.ds(cc, 128)],
            tile_ref.at[i], sem,
        ).start()
        return 0

    lax.fori_loop(0, n, issue, 0)

    def drain(i, _):
        pltpu.make_async_copy(
            out_hbm.at[pl.ds(0, 8), pl.ds(0, 128)], tile_ref.at[0], sem
        ).wait()
        return 0

    lax.fori_loop(0, n, drain, 0)

    tiles = tile_ref[...]
    ii = lax.broadcasted_iota(jnp.int32, (n, 8, 128), 0)
    rr = lax.broadcasted_iota(jnp.int32, (n, 8, 128), 1)
    rows = jnp.where(rr == (ii // _NUM_TRUE) % 8, tiles, 0.0)
    picked = jnp.sum(rows, axis=1).reshape(_BATCH, _NUM_TRUE, 128)
    tgtv = tgtv_ref[...]
    ll = lax.broadcasted_iota(jnp.int32, (_BATCH, _NUM_TRUE, 128), 2)
    lane = tgtv - jnp.minimum((tgtv // 128) * 128, _NUM_CLASSES - 128)
    true_logits = jnp.sum(
        jnp.where(ll == lane[:, :, None], picked, 0.0), axis=2)

    tcorr = _log_expected_count(tgtv.astype(jnp.float32))
    adj_t = true_logits - tcorr
    total = s_ref[...] + jnp.sum(jnp.exp(adj_t), axis=1, keepdims=True)
    loss_b = jnp.log(total) - jnp.mean(adj_t, axis=1, keepdims=True)
    loss_ref[...] = jnp.mean(loss_b).reshape(1, 1)


def _loss(s_acc, target, out):
    return pl.pallas_call(
        _loss_body,
        in_specs=[
            pl.BlockSpec((_BATCH, 1), lambda: (0, 0)),
            pl.BlockSpec(memory_space=pltpu.SMEM),
            pl.BlockSpec((_BATCH, _NUM_TRUE), lambda: (0, 0)),
            pl.BlockSpec(memory_space=pl.ANY),
        ],
        out_specs=[
            pl.BlockSpec((1, 1), lambda: (0, 0)),
            pl.BlockSpec(memory_space=pl.ANY),
        ],
        out_shape=[
            jax.ShapeDtypeStruct((1, 1), jnp.float32),
            jax.ShapeDtypeStruct((_BATCH, _NUM_CLASSES), jnp.float32),
        ],
        input_output_aliases={3: 1},
        scratch_shapes=[
            pltpu.VMEM((_BATCH * _NUM_TRUE, 8, 128), jnp.float32),
            pltpu.SemaphoreType.DMA,
        ],
    )(s_acc, target, target, out)


def _wvec_const():
    with jax.default_device(jax.devices("cpu")[0]):
        u = jax.random.uniform(jax.random.key(42), (_NUM_SAMPLED,),
                               dtype=jnp.float32)
        sampled = jnp.clip(
            (jnp.exp(u * jnp.log(_NUM_CLASSES + 1.0)) - 1.0).astype(jnp.int32),
            0, _NUM_CLASSES - 1)
        inv_exp = jnp.exp(-_log_expected_count(sampled.astype(jnp.float32)))
        wvec = jnp.zeros((_NUM_CLASSES,), jnp.float32).at[sampled].add(inv_exp)
        return np.asarray(wvec).reshape(1, _NUM_CLASSES)


_WVEC = _wvec_const()


def kernel(pred, kernel, bias, target):
    wvec = jnp.asarray(_WVEC)
    out, s_acc = _projection(pred, kernel, bias.reshape(1, _NUM_CLASSES),
                             wvec)
    loss, out = _loss(s_acc, target, out)
    return out, loss.reshape(())


# speedup vs baseline: 2.4149x; 2.4149x over previous
import jax
import jax.numpy as jnp
import numpy as np
from jax import lax
from jax.experimental import pallas as pl
from jax.experimental.pallas import tpu as pltpu

_NUM_CLASSES = 100000
_NUM_SAMPLED = 8192
_NUM_TRUE = 3
_DIM = 128
_BATCH = 1024

_BN = 2000
_NBLK = _NUM_CLASSES // _BN


def _log_expected_count(ids_f32):
    p = (jnp.log(ids_f32 + 2.0) - jnp.log(ids_f32 + 1.0)) / jnp.log(
        jnp.float32(_NUM_CLASSES + 1.0))
    return jnp.log(1.0 - jnp.exp(_NUM_SAMPLED * jnp.log(1.0 - p)))


def _wvec_const():
    with jax.default_device(jax.devices("cpu")[0]):
        u = jax.random.uniform(jax.random.key(42), (_NUM_SAMPLED,),
                               dtype=jnp.float32)
        sampled = jnp.clip(
            (jnp.exp(u * jnp.log(_NUM_CLASSES + 1.0)) - 1.0).astype(jnp.int32),
            0, _NUM_CLASSES - 1)
        inv_exp = jnp.exp(-_log_expected_count(sampled.astype(jnp.float32)))
        wvec = jnp.zeros((_NUM_CLASSES,), jnp.float32).at[sampled].add(inv_exp)
        return np.asarray(wvec).reshape(_NUM_CLASSES, 1)


_WVEC = _wvec_const()


def _proj_body(kt_ref, pred_ref, b_ref, wv_ref, out_ref, s_ref, acc_ref):
    k = pl.program_id(0)
    tile = lax.dot_general(
        kt_ref[...], pred_ref[...], (((1,), (1,)), ((), ())),
        preferred_element_type=jnp.float32) + b_ref[...]
    out_ref[...] = tile
    weighted = wv_ref[...] * jnp.exp(tile)

    @pl.when(k == 0)
    def _init():
        acc_ref[...] = jnp.zeros_like(acc_ref)

    acc_ref[...] += jnp.sum(weighted, axis=0, keepdims=True)

    @pl.when(k == _NBLK - 1)
    def _finish():
        s_ref[...] = acc_ref[...]


def _projection(kt, pred, bias2, wvec2):
    return pl.pallas_call(
        _proj_body,
        grid=(_NBLK,),
        in_specs=[
            pl.BlockSpec((_BN, _DIM), lambda k: (k, 0)),
            pl.BlockSpec((_BATCH, _DIM), lambda k: (0, 0)),
            pl.BlockSpec((_BN, 1), lambda k: (k, 0)),
            pl.BlockSpec((_BN, 1), lambda k: (k, 0)),
        ],
        out_specs=[
            pl.BlockSpec((_BN, _BATCH), lambda k: (k, 0)),
            pl.BlockSpec((1, _BATCH), lambda k: (0, 0)),
        ],
        out_shape=[
            jax.ShapeDtypeStruct((_NUM_CLASSES, _BATCH), jnp.float32),
            jax.ShapeDtypeStruct((1, _BATCH), jnp.float32),
        ],
        scratch_shapes=[pltpu.VMEM((1, _BATCH), jnp.float32)],
    )(kt, pred, bias2, wvec2)


def _loss_body(s_ref, tgt_ref, tgtv_ref, out_hbm, loss_ref, out_alias_ref,
               tile_ref, sem):
    del out_alias_ref
    n = _BATCH * _NUM_TRUE

    def issue(b, _):
        cb = pl.multiple_of((b // 128) * 128, 128)
        for t in range(_NUM_TRUE):
            cls = tgt_ref[t, b]
            rb = pl.multiple_of((cls // 8) * 8, 8)
            pltpu.make_async_copy(
                out_hbm.at[pl.ds(rb, 8), pl.ds(cb, 128)],
                tile_ref.at[t * _BATCH + b], sem,
            ).start()
        return 0

    lax.fori_loop(0, _BATCH, issue, 0)

    def drain(i, _):
        pltpu.make_async_copy(
            out_hbm.at[pl.ds(0, 8), pl.ds(0, 128)], tile_ref.at[0], sem
        ).wait()
        return 0

    lax.fori_loop(0, n, drain, 0)

    tiles = tile_ref[...]
    ii = lax.broadcasted_iota(jnp.int32, (n, 8, 128), 0)
    ll = lax.broadcasted_iota(jnp.int32, (n, 8, 128), 2)
    s1 = jnp.sum(jnp.where(ll == ii % 128, tiles, 0.0), axis=2)
    s1 = s1.reshape(_NUM_TRUE, _BATCH, 8)
    tgtv = tgtv_ref[...]
    rr = lax.broadcasted_iota(jnp.int32, (_NUM_TRUE, _BATCH, 8), 2)
    true_logits = jnp.sum(
        jnp.where(rr == (tgtv % 8)[:, :, None], s1, 0.0), axis=2)

    tcorr = _log_expected_count(tgtv.astype(jnp.float32))
    adj_t = true_logits - tcorr
    total = s_ref[...] + jnp.sum(jnp.exp(adj_t), axis=0, keepdims=True)
    loss_b = jnp.log(total) - jnp.mean(adj_t, axis=0, keepdims=True)
    loss_ref[...] = jnp.mean(loss_b).reshape(1, 1)


def _loss(s_acc, tgtT, out_phys):
    return pl.pallas_call(
        _loss_body,
        in_specs=[
            pl.BlockSpec((1, _BATCH), lambda: (0, 0)),
            pl.BlockSpec(memory_space=pltpu.SMEM),
            pl.BlockSpec((_NUM_TRUE, _BATCH), lambda: (0, 0)),
            pl.BlockSpec(memory_space=pl.ANY),
        ],
        out_specs=[
            pl.BlockSpec((1, 1), lambda: (0, 0)),
            pl.BlockSpec(memory_space=pl.ANY),
        ],
        out_shape=[
            jax.ShapeDtypeStruct((1, 1), jnp.float32),
            jax.ShapeDtypeStruct((_NUM_CLASSES, _BATCH), jnp.float32),
        ],
        input_output_aliases={3: 1},
        scratch_shapes=[
            pltpu.VMEM((_BATCH * _NUM_TRUE, 8, 128), jnp.float32),
            pltpu.SemaphoreType.DMA,
        ],
    )(s_acc, tgtT, tgtT, out_phys)


def kernel(pred, kernel, bias, target):
    kt = kernel.T
    tgtT = target.T
    wvec = jnp.asarray(_WVEC)
    out_phys, s_acc = _projection(kt, pred, bias.reshape(_NUM_CLASSES, 1),
                                  wvec)
    loss, out_phys = _loss(s_acc, tgtT, out_phys)
    return out_phys.T, loss.reshape(())


# speedup vs baseline: 2.6277x; 1.0881x over previous
import jax
import jax.numpy as jnp
import numpy as np
from jax import lax
from jax.experimental import pallas as pl
from jax.experimental.pallas import tpu as pltpu

_NUM_CLASSES = 100000
_NUM_SAMPLED = 8192
_NUM_TRUE = 3
_DIM = 128
_BATCH = 1024

_BN = 2000
_NBLK = _NUM_CLASSES // _BN


def _log_expected_count(ids_f32):
    p = (jnp.log(ids_f32 + 2.0) - jnp.log(ids_f32 + 1.0)) / jnp.log(
        jnp.float32(_NUM_CLASSES + 1.0))
    return jnp.log(1.0 - jnp.exp(_NUM_SAMPLED * jnp.log(1.0 - p)))


def _threefry2x32_np(k1, k2, x0, x1):
    def rotl(x, d):
        return ((x << np.uint32(d)) | (x >> np.uint32(32 - d))).astype(np.uint32)

    rotations = ((13, 15, 26, 6), (17, 29, 16, 24))
    ks = [np.uint32(k1), np.uint32(k2),
          np.uint32(k1 ^ k2 ^ np.uint32(0x1BD11BDA))]
    x0 = (x0 + ks[0]).astype(np.uint32)
    x1 = (x1 + ks[1]).astype(np.uint32)
    for i in range(5):
        for r in rotations[i % 2]:
            x0 = (x0 + x1).astype(np.uint32)
            x1 = (x0 ^ rotl(x1, r)).astype(np.uint32)
        x0 = (x0 + ks[(i + 1) % 3]).astype(np.uint32)
        x1 = (x1 + ks[(i + 2) % 3] + np.uint32(i + 1)).astype(np.uint32)
    return x0, x1


def _wvec_const():
    hi = np.zeros(_NUM_SAMPLED, dtype=np.uint32)
    lo = np.arange(_NUM_SAMPLED, dtype=np.uint32)
    b0, b1 = _threefry2x32_np(np.uint32(0), np.uint32(42), hi, lo)
    bits = (b0 ^ b1).astype(np.uint32)
    u = ((bits >> np.uint32(9)) | np.uint32(0x3F800000)).view(np.float32) \
        - np.float32(1.0)
    sampled = np.clip(
        (np.exp(u * np.float32(np.log(np.float32(_NUM_CLASSES + 1.0))))
         - np.float32(1.0)).astype(np.int32), 0, _NUM_CLASSES - 1)
    ids = sampled.astype(np.float64)
    p = (np.log(ids + 2.0) - np.log(ids + 1.0)) / np.log(_NUM_CLASSES + 1.0)
    expected = -np.expm1(_NUM_SAMPLED * np.log1p(-p))
    wvec = np.zeros(_NUM_CLASSES, dtype=np.float64)
    np.add.at(wvec, sampled, 1.0 / expected)
    return wvec.astype(np.float32).reshape(_NUM_CLASSES, 1)


_WVEC = _wvec_const()


def _proj_body(kt_ref, pred_ref, b_ref, wv_ref, out_ref, s_ref, acc_ref):
    k = pl.program_id(0)
    tile = lax.dot_general(
        kt_ref[...], pred_ref[...], (((1,), (1,)), ((), ())),
        preferred_element_type=jnp.float32) + b_ref[...]
    out_ref[...] = tile
    contrib = lax.dot_general(
        wv_ref[...], jnp.exp(tile), (((0,), (0,)), ((), ())),
        preferred_element_type=jnp.float32)

    @pl.when(k == 0)
    def _init():
        acc_ref[...] = jnp.zeros_like(acc_ref)

    acc_ref[...] += contrib

    @pl.when(k == _NBLK - 1)
    def _finish():
        s_ref[...] = acc_ref[...]


def _projection(kt, pred, bias2, wvec2):
    return pl.pallas_call(
        _proj_body,
        grid=(_NBLK,),
        in_specs=[
            pl.BlockSpec((_BN, _DIM), lambda k: (k, 0)),
            pl.BlockSpec((_BATCH, _DIM), lambda k: (0, 0)),
            pl.BlockSpec((_BN, 1), lambda k: (k, 0)),
            pl.BlockSpec((_BN, 1), lambda k: (k, 0)),
        ],
        out_specs=[
            pl.BlockSpec((_BN, _BATCH), lambda k: (k, 0)),
            pl.BlockSpec((1, _BATCH), lambda k: (0, 0)),
        ],
        out_shape=[
            jax.ShapeDtypeStruct((_NUM_CLASSES, _BATCH), jnp.float32),
            jax.ShapeDtypeStruct((1, _BATCH), jnp.float32),
        ],
        scratch_shapes=[pltpu.VMEM((1, _BATCH), jnp.float32)],
    )(kt, pred, bias2, wvec2)


def _loss_body(s_ref, tgt_ref, tgtv_ref, out_hbm, loss_ref, out_alias_ref,
               tile_ref, sem):
    del out_alias_ref
    n = _BATCH * _NUM_TRUE

    def issue(g, _):
        cb = pl.multiple_of((g // 16) * 128, 128)
        for j in range(8):
            b = g * 8 + j
            for t in range(_NUM_TRUE):
                cls = tgt_ref[t, b]
                rb = pl.multiple_of((cls // 8) * 8, 8)
                pltpu.make_async_copy(
                    out_hbm.at[pl.ds(rb, 8), pl.ds(cb, 128)],
                    tile_ref.at[t * _BATCH + b], sem.at[j],
                ).start()
        return 0

    lax.fori_loop(0, _BATCH // 8, issue, 0)

    def drain(i, _):
        for j in range(8):
            pltpu.make_async_copy(
                out_hbm.at[pl.ds(0, 8), pl.ds(0, 128)], tile_ref.at[0],
                sem.at[j],
            ).wait()
        return 0

    lax.fori_loop(0, n // 8, drain, 0)

    tiles = tile_ref[...]
    ii = lax.broadcasted_iota(jnp.int32, (n, 8, 128), 0)
    ll = lax.broadcasted_iota(jnp.int32, (n, 8, 128), 2)
    s1 = jnp.sum(jnp.where(ll == ii % 128, tiles, 0.0), axis=2)
    s1 = s1.reshape(_NUM_TRUE, _BATCH, 8)
    tgtv = tgtv_ref[...]
    rr = lax.broadcasted_iota(jnp.int32, (_NUM_TRUE, _BATCH, 8), 2)
    true_logits = jnp.sum(
        jnp.where(rr == (tgtv % 8)[:, :, None], s1, 0.0), axis=2)

    tcorr = _log_expected_count(tgtv.astype(jnp.float32))
    adj_t = true_logits - tcorr
    total = s_ref[...] + jnp.sum(jnp.exp(adj_t), axis=0, keepdims=True)
    loss_b = jnp.log(total) - jnp.mean(adj_t, axis=0, keepdims=True)
    loss_ref[...] = jnp.mean(loss_b).reshape(1, 1)


def _loss(s_acc, tgtT, out_phys):
    return pl.pallas_call(
        _loss_body,
        in_specs=[
            pl.BlockSpec((1, _BATCH), lambda: (0, 0)),
            pl.BlockSpec(memory_space=pltpu.SMEM),
            pl.BlockSpec((_NUM_TRUE, _BATCH), lambda: (0, 0)),
            pl.BlockSpec(memory_space=pl.ANY),
        ],
        out_specs=[
            pl.BlockSpec((1, 1), lambda: (0, 0)),
            pl.BlockSpec(memory_space=pl.ANY),
        ],
        out_shape=[
            jax.ShapeDtypeStruct((1, 1), jnp.float32),
            jax.ShapeDtypeStruct((_NUM_CLASSES, _BATCH), jnp.float32),
        ],
        input_output_aliases={3: 1},
        scratch_shapes=[
            pltpu.VMEM((_BATCH * _NUM_TRUE, 8, 128), jnp.float32),
            pltpu.SemaphoreType.DMA((8,)),
        ],
    )(s_acc, tgtT, tgtT, out_phys)


def kernel(pred, kernel, bias, target):
    kt = kernel.T
    tgtT = target.T
    wvec = jnp.asarray(_WVEC)
    out_phys, s_acc = _projection(kt, pred, bias.reshape(_NUM_CLASSES, 1),
                                  wvec)
    loss, out_phys = _loss(s_acc, tgtT, out_phys)
    return out_phys.T, loss.reshape(())
